# baseline (speedup 1.0000x reference)
import functools

import jax
import jax.numpy as jnp
from jax import lax
from jax.experimental import pallas as pl
from jax.experimental.pallas import tpu as pltpu
from jax.experimental.pallas import tpu_sc as plsc

B = 4096
L = 50
LPAD = 56
D = 128
NUM_WORKERS = 32
RPW = B // NUM_WORKERS
IDX_PER_W = RPW * LPAD
REVIEWS_PER_STEP = 2
CHUNK = REVIEWS_PER_STEP * LPAD
NSTEPS = RPW // REVIEWS_PER_STEP
LANES = 16


def _pool_body(idx_hbm, table_hbm, out_hbm, idx_v, rows_v, out_v):
    wid = lax.axis_index("s") * 2 + lax.axis_index("c")
    base = wid * IDX_PER_W
    pltpu.sync_copy(idx_hbm.at[pl.ds(base, IDX_PER_W)], idx_v)

    @pl.loop(0, NSTEPS)
    def _(k):
        pltpu.sync_copy(table_hbm.at[idx_v.at[pl.ds(k * CHUNK, CHUNK)]], rows_v)

        def accum(w, carry):
            new = []
            for r2 in range(REVIEWS_PER_STEP):
                for c in range(D // LANES):
                    v = rows_v[r2 * LPAD + w, pl.ds(c * LANES, LANES)]
                    new.append(carry[r2 * (D // LANES) + c] + v)
            return tuple(new)

        init = tuple(
            rows_v[r2 * LPAD, pl.ds(c * LANES, LANES)]
            for r2 in range(REVIEWS_PER_STEP)
            for c in range(D // LANES)
        )
        acc = lax.fori_loop(1, L, accum, init)
        for r2 in range(REVIEWS_PER_STEP):
            for c in range(D // LANES):
                out_v[k * REVIEWS_PER_STEP + r2, pl.ds(c * LANES, LANES)] = (
                    acc[r2 * (D // LANES) + c] * (1.0 / L)
                )

    pltpu.sync_copy(out_v, out_hbm.at[pl.ds(wid * RPW, RPW)])


def _sc_pool(idx_flat, table):
    kern = functools.partial(
        pl.kernel,
        out_type=jax.ShapeDtypeStruct((B, D), jnp.float32),
        mesh=plsc.VectorSubcoreMesh(core_axis_name="c", subcore_axis_name="s"),
        scratch_types=[
            pltpu.VMEM((IDX_PER_W,), jnp.int32),
            pltpu.VMEM((CHUNK, D), jnp.float32),
            pltpu.VMEM((RPW, D), jnp.float32),
        ],
    )(_pool_body)
    return kern(idx_flat, table)


def _mlp_body(x_ref, w1_ref, b1_ref, w2_ref, b2_ref, w3_ref, b3_ref, o_ref):
    x = x_ref[...]
    h = jnp.dot(x, w1_ref[...], preferred_element_type=jnp.float32) + b1_ref[...]
    h = jnp.maximum(h, 0.0)
    h = jnp.dot(h, w2_ref[...], preferred_element_type=jnp.float32) + b2_ref[...]
    h = jnp.maximum(h, 0.0)
    logits = jnp.dot(h, w3_ref[...], preferred_element_type=jnp.float32) + b3_ref[...]
    m = jnp.max(logits, axis=-1, keepdims=True)
    e = jnp.exp(logits - m)
    o_ref[...] = e / jnp.sum(e, axis=-1, keepdims=True)


def _tc_mlp(pooled, W1, b1, W2, b2, W3, b3):
    return pl.pallas_call(
        _mlp_body,
        out_shape=jax.ShapeDtypeStruct((B, 2), jnp.float32),
    )(pooled, W1, b1.reshape(1, -1), W2, b2.reshape(1, -1), W3, b3.reshape(1, -1))


@jax.jit
def kernel(indices, table, W1, b1, W2, b2, W3, b3):
    idx = indices.astype(jnp.int32)
    idx_pad = jnp.pad(idx, ((0, 0), (0, LPAD - L))).reshape(B * LPAD)
    pooled = _sc_pool(idx_pad, table)
    return _tc_mlp(pooled, W1, b1, W2, b2, W3, b3)
